# Initial kernel scaffold; baseline (speedup 1.0000x reference)
import jax
import jax.numpy as jnp
from jax.experimental import pallas as pl


def kernel(x, W_enc, b_enc, W_dec, b_dec):
    raise NotImplementedError("write your pallas kernel here")


# speedup vs baseline: 1.5451x; 1.5451x over previous
import functools

import jax
import jax.numpy as jnp
from jax import lax
from jax.experimental import pallas as pl
from jax.experimental.pallas import tpu as pltpu

B = 1024
D_IN = 2048
NUM_LATENTS = 32768
K = 32
BLK = 1024
NB = NUM_LATENTS // BLK

_NEG = float('-inf')


def _encode_body(x_ref, bdec_ref, w_ref, benc_ref, tv_out, ti_out, tv, ti):
    j = pl.program_id(0)

    @pl.when(j == 0)
    def _init():
        tv[...] = jnp.full((B, K), _NEG, jnp.float32)
        ti[...] = jnp.zeros((B, K), jnp.int32)

    xm = x_ref[...] - bdec_ref[...]
    pre = lax.dot_general(
        xm, w_ref[...], (((1,), (1,)), ((), ())),
        preferred_element_type=jnp.float32,
    ) + benc_ref[...]

    lane = lax.broadcasted_iota(jnp.int32, (B, BLK), 1)
    kl = lax.broadcasted_iota(jnp.int32, (B, K), 1)
    work = pre
    for _ in range(K):
        m = jnp.max(work, axis=1, keepdims=True)
        am = jnp.min(jnp.where(work == m, lane, BLK), axis=1, keepdims=True)
        tvv = tv[...]
        rmin = jnp.min(tvv, axis=1, keepdims=True)
        rpos = jnp.min(jnp.where(tvv == rmin, kl, K), axis=1, keepdims=True)
        do = m > rmin
        sel = (kl == rpos) & do
        tv[...] = jnp.where(sel, m, tvv)
        ti[...] = jnp.where(sel, am + j * BLK, ti[...])
        work = jnp.where(lane == am, _NEG, work)

    @pl.when(j == NB - 1)
    def _fin():
        tv_out[...] = tv[...]
        ti_out[...] = ti[...]


def _encode_topk(x, W_enc, b_enc, b_dec):
    benc2 = b_enc.reshape(1, NUM_LATENTS)
    bdec2 = b_dec.reshape(1, D_IN)
    return pl.pallas_call(
        _encode_body,
        grid=(NB,),
        in_specs=[
            pl.BlockSpec((B, D_IN), lambda j: (0, 0)),
            pl.BlockSpec((1, D_IN), lambda j: (0, 0)),
            pl.BlockSpec((BLK, D_IN), lambda j: (j, 0)),
            pl.BlockSpec((1, BLK), lambda j: (0, j)),
        ],
        out_specs=[
            pl.BlockSpec((B, K), lambda j: (0, 0)),
            pl.BlockSpec((B, K), lambda j: (0, 0)),
        ],
        out_shape=[
            jax.ShapeDtypeStruct((B, K), jnp.float32),
            jax.ShapeDtypeStruct((B, K), jnp.int32),
        ],
        scratch_shapes=[
            pltpu.VMEM((B, K), jnp.float32),
            pltpu.VMEM((B, K), jnp.int32),
        ],
    )(x, bdec2, W_enc, benc2)


def kernel(x, W_enc, b_enc, W_dec, b_dec):
    top_acts, top_idx = _encode_topk(x, W_enc, b_enc, b_dec)
    gathered = jnp.take(W_dec, top_idx, axis=0)
    res = jnp.einsum('bk,bkd->bd', top_acts, gathered)
    return res + b_dec


# speedup vs baseline: 3.3932x; 2.1961x over previous
import functools

import jax
import jax.numpy as jnp
from jax import lax
from jax.experimental import pallas as pl
from jax.experimental.pallas import tpu as pltpu
from jax.experimental.pallas import tpu_sc as plsc

B = 1024
D_IN = 2048
NUM_LATENTS = 32768
K = 32
BLK = 1024
NB = NUM_LATENTS // BLK

_NEG = float('-inf')


def _encode_body(x_ref, bdec_ref, w_ref, benc_ref, tv_out, ti_out, tv, ti):
    j = pl.program_id(0)

    @pl.when(j == 0)
    def _init():
        tv[...] = jnp.full((B, K), _NEG, jnp.float32)
        ti[...] = jnp.zeros((B, K), jnp.int32)

    xm = x_ref[...] - bdec_ref[...]
    pre = lax.dot_general(
        xm, w_ref[...], (((1,), (1,)), ((), ())),
        preferred_element_type=jnp.float32,
    ) + benc_ref[...]

    lane = lax.broadcasted_iota(jnp.int32, (B, BLK), 1)
    kl = lax.broadcasted_iota(jnp.int32, (B, K), 1)

    m0 = jnp.max(pre, axis=1, keepdims=True)
    rmin0 = jnp.min(tv[...], axis=1, keepdims=True)

    def _cond(c):
        return c[2]

    def _body(c):
        work, m, _ = c
        am = jnp.min(jnp.where(work == m, lane, BLK), axis=1, keepdims=True)
        tvv = tv[...]
        rmin = jnp.min(tvv, axis=1, keepdims=True)
        rpos = jnp.min(jnp.where(tvv == rmin, kl, K), axis=1, keepdims=True)
        do = m > rmin
        sel = (kl == rpos) & do
        tv[...] = jnp.where(sel, m, tvv)
        ti[...] = jnp.where(sel, am + j * BLK, ti[...])
        work = jnp.where((lane == am) & do, _NEG, work)
        m2 = jnp.max(work, axis=1, keepdims=True)
        rmin2 = jnp.min(tv[...], axis=1, keepdims=True)
        return (work, m2, jnp.any(m2 > rmin2))

    lax.while_loop(_cond, _body, (pre, m0, jnp.any(m0 > rmin0)))

    @pl.when(j == NB - 1)
    def _fin():
        tv_out[...] = tv[...]
        ti_out[...] = ti[...]


def _encode_topk(x, W_enc, b_enc, b_dec):
    benc2 = b_enc.reshape(1, NUM_LATENTS)
    bdec2 = b_dec.reshape(1, D_IN)
    return pl.pallas_call(
        _encode_body,
        grid=(NB,),
        in_specs=[
            pl.BlockSpec((B, D_IN), lambda j: (0, 0)),
            pl.BlockSpec((1, D_IN), lambda j: (0, 0)),
            pl.BlockSpec((BLK, D_IN), lambda j: (j, 0)),
            pl.BlockSpec((1, BLK), lambda j: (0, j)),
        ],
        out_specs=[
            pl.BlockSpec((B, K), lambda j: (0, 0)),
            pl.BlockSpec((B, K), lambda j: (0, 0)),
        ],
        out_shape=[
            jax.ShapeDtypeStruct((B, K), jnp.float32),
            jax.ShapeDtypeStruct((B, K), jnp.int32),
        ],
        scratch_shapes=[
            pltpu.VMEM((B, K), jnp.float32),
            pltpu.VMEM((B, K), jnp.int32),
        ],
    )(x, bdec2, W_enc, benc2)


def _make_decode():
    info = plsc.get_sparse_core_info()
    NC, NS, L = info.num_cores, info.num_subcores, info.num_lanes
    NW = NC * NS
    BPW = B // NW
    mesh = plsc.VectorSubcoreMesh(core_axis_name="c", subcore_axis_name="s")

    @functools.partial(
        pl.kernel, mesh=mesh,
        out_type=jax.ShapeDtypeStruct((B * D_IN,), jnp.float32),
        scratch_types=[
            pltpu.VMEM((K,), jnp.int32),
            pltpu.VMEM((K, L), jnp.float32),
            pltpu.VMEM((K, D_IN), jnp.float32),
            pltpu.VMEM((D_IN,), jnp.float32),
            pltpu.VMEM((D_IN,), jnp.float32),
            pltpu.SemaphoreType.DMA,
        ],
    )
    def _dec(acts_hbm, idx_hbm, wdec_hbm, bdec_hbm, out_hbm,
             idx_v, w_v, rows_v, bdec_v, acc_v, sem):
        wid = lax.axis_index("s") * NC + lax.axis_index("c")
        pltpu.sync_copy(bdec_hbm, bdec_v)

        def b_body(i, carry):
            b = wid * BPW + i
            pltpu.sync_copy(idx_hbm.at[pl.ds(b * K, K)], idx_v)
            pltpu.sync_copy(acts_hbm.at[pl.ds(b * K, K)], w_v)
            pltpu.async_copy(wdec_hbm.at[idx_v], rows_v, sem).wait()

            def c_body(c, carry2):
                off = c * L
                acc = bdec_v[pl.ds(off, L)]
                for k in range(K):
                    acc = acc + w_v[k] * rows_v[k, pl.ds(off, L)]
                acc_v[pl.ds(off, L)] = acc
                return carry2

            lax.fori_loop(0, D_IN // L, c_body, 0)
            pltpu.sync_copy(acc_v, out_hbm.at[pl.ds(b * D_IN, D_IN)])
            return carry

        lax.fori_loop(0, BPW, b_body, 0)

    return _dec


def kernel(x, W_enc, b_enc, W_dec, b_dec):
    top_acts, top_idx = _encode_topk(x, W_enc, b_enc, b_dec)
    dec = _make_decode()
    acts16 = jnp.broadcast_to(
        top_acts.reshape(B * K, 1), (B * K, 16)).astype(jnp.float32)
    res_flat = dec(acts16, top_idx.reshape(B * K), W_dec, b_dec)
    return res_flat.reshape(B, D_IN)


# speedup vs baseline: 3.6809x; 1.0848x over previous
import functools

import jax
import jax.numpy as jnp
from jax import lax
from jax.experimental import pallas as pl
from jax.experimental.pallas import tpu as pltpu
from jax.experimental.pallas import tpu_sc as plsc

B = 1024
D_IN = 2048
NUM_LATENTS = 32768
K = 32
BLK = 1024
NB = NUM_LATENTS // BLK

_NEG = float('-inf')


def _encode_body(x_ref, bdec_ref, w_ref, benc_ref, tv_out, ti_out, tv, ti):
    j = pl.program_id(0)

    @pl.when(j == 0)
    def _init():
        tv[...] = jnp.full((B, K), _NEG, jnp.float32)
        ti[...] = jnp.zeros((B, K), jnp.int32)

    xm = x_ref[...] - bdec_ref[...]
    pre = lax.dot_general(
        xm, w_ref[...], (((1,), (1,)), ((), ())),
        preferred_element_type=jnp.float32,
    ) + benc_ref[...]

    lane = lax.broadcasted_iota(jnp.int32, (B, BLK), 1)
    kl = lax.broadcasted_iota(jnp.int32, (B, K), 1)

    m0 = jnp.max(pre, axis=1, keepdims=True)
    rmin0 = jnp.min(tv[...], axis=1, keepdims=True)

    def _cond(c):
        return c[2]

    def _body(c):
        work, m, _ = c
        am = jnp.min(jnp.where(work == m, lane, BLK), axis=1, keepdims=True)
        tvv = tv[...]
        rmin = jnp.min(tvv, axis=1, keepdims=True)
        rpos = jnp.min(jnp.where(tvv == rmin, kl, K), axis=1, keepdims=True)
        do = m > rmin
        sel = (kl == rpos) & do
        tv[...] = jnp.where(sel, m, tvv)
        ti[...] = jnp.where(sel, am + j * BLK, ti[...])
        work = jnp.where((lane == am) & do, _NEG, work)
        m2 = jnp.max(work, axis=1, keepdims=True)
        rmin2 = jnp.min(tv[...], axis=1, keepdims=True)
        return (work, m2, jnp.any(m2 > rmin2))

    lax.while_loop(_cond, _body, (pre, m0, jnp.any(m0 > rmin0)))

    @pl.when(j == NB - 1)
    def _fin():
        tv_out[...] = tv[...]
        ti_out[...] = ti[...]


def _encode_topk(x, W_enc, b_enc, b_dec):
    benc2 = b_enc.reshape(1, NUM_LATENTS)
    bdec2 = b_dec.reshape(1, D_IN)
    return pl.pallas_call(
        _encode_body,
        grid=(NB,),
        in_specs=[
            pl.BlockSpec((B, D_IN), lambda j: (0, 0)),
            pl.BlockSpec((1, D_IN), lambda j: (0, 0)),
            pl.BlockSpec((BLK, D_IN), lambda j: (j, 0)),
            pl.BlockSpec((1, BLK), lambda j: (0, j)),
        ],
        out_specs=[
            pl.BlockSpec((B, K), lambda j: (0, 0)),
            pl.BlockSpec((B, K), lambda j: (0, 0)),
        ],
        out_shape=[
            jax.ShapeDtypeStruct((B, K), jnp.float32),
            jax.ShapeDtypeStruct((B, K), jnp.int32),
        ],
        scratch_shapes=[
            pltpu.VMEM((B, K), jnp.float32),
            pltpu.VMEM((B, K), jnp.int32),
        ],
    )(x, bdec2, W_enc, benc2)


def _make_decode():
    info = plsc.get_sparse_core_info()
    NC, NS, L = info.num_cores, info.num_subcores, info.num_lanes
    NW = NC * NS
    BPW = B // NW
    mesh = plsc.VectorSubcoreMesh(core_axis_name="c", subcore_axis_name="s")

    KH = K // 2

    @functools.partial(
        pl.kernel, mesh=mesh,
        out_type=jax.ShapeDtypeStruct((B * D_IN,), jnp.float32),
        scratch_types=[
            pltpu.VMEM((BPW * K,), jnp.int32),
            pltpu.VMEM((BPW * K * L,), jnp.float32),
            pltpu.VMEM((KH, D_IN), jnp.float32),
            pltpu.VMEM((KH, D_IN), jnp.float32),
            pltpu.VMEM((D_IN,), jnp.float32),
            pltpu.VMEM((D_IN,), jnp.float32),
            pltpu.SemaphoreType.DMA,
            pltpu.SemaphoreType.DMA,
        ],
    )
    def _dec(acts_hbm, idx_hbm, wdec_hbm, bdec_hbm, out_hbm,
             idx_v, w_v, rows0_v, rows1_v, bdec_v, acc_v, sem0, sem1):
        wid = lax.axis_index("s") * NC + lax.axis_index("c")
        base = wid * BPW * K
        pltpu.sync_copy(bdec_hbm, bdec_v)
        pltpu.sync_copy(idx_hbm.at[pl.ds(base, BPW * K)], idx_v)
        pltpu.sync_copy(acts_hbm.at[pl.ds(base * L, BPW * K * L)], w_v)

        rows = (rows0_v, rows1_v)
        sems = (sem0, sem1)

        def _issue(s, p):
            pltpu.async_copy(
                wdec_hbm.at[idx_v.at[pl.ds(s * KH, KH)]], rows[p], sems[p])

        def _drain(p):
            pltpu.make_async_copy(
                wdec_hbm.at[pl.ds(0, KH)], rows[p], sems[p]).wait()

        _issue(0, 0)
        _issue(1, 1)

        def b_body(i, carry):
            for h in (0, 1):
                s = 2 * i + h
                _drain(h)
                ws = [w_v[pl.ds((s * KH + k) * L, L)] for k in range(KH)]

                def c_body(c, carry2, h=h, ws=ws):
                    off = c * L
                    if h == 0:
                        acc = bdec_v[pl.ds(off, L)]
                    else:
                        acc = acc_v[pl.ds(off, L)]
                    r = rows[h]
                    for k in range(KH):
                        acc = acc + ws[k] * r[k, pl.ds(off, L)]
                    acc_v[pl.ds(off, L)] = acc
                    return carry2

                lax.fori_loop(0, D_IN // L, c_body, 0)

                @pl.when(i + 1 < BPW)
                def _next(s=s, h=h):
                    _issue(s + 2, h)

            pltpu.sync_copy(
                acc_v, out_hbm.at[pl.ds((wid * BPW + i) * D_IN, D_IN)])
            return carry

        lax.fori_loop(0, BPW, b_body, 0)

    return _dec


def kernel(x, W_enc, b_enc, W_dec, b_dec):
    top_acts, top_idx = _encode_topk(x, W_enc, b_enc, b_dec)
    dec = _make_decode()
    acts16 = jnp.broadcast_to(
        top_acts.reshape(B * K, 1), (B * K, 16)).reshape(-1).astype(jnp.float32)
    res_flat = dec(acts16, top_idx.reshape(B * K), W_dec, b_dec)
    return res_flat.reshape(B, D_IN)


# speedup vs baseline: 3.7503x; 1.0188x over previous
import functools

import jax
import jax.numpy as jnp
from jax import lax
from jax.experimental import pallas as pl
from jax.experimental.pallas import tpu as pltpu
from jax.experimental.pallas import tpu_sc as plsc

B = 1024
D_IN = 2048
NUM_LATENTS = 32768
K = 32
BLK = 1024
NB = NUM_LATENTS // BLK

_NEG = float('-inf')


def _encode_body(x_ref, bdec_ref, w_ref, benc_ref, tv_out, ti_out, tv, ti,
                 *, bc):
    j = pl.program_id(0)

    @pl.when(j == 0)
    def _init():
        tv[...] = jnp.full((bc, K), _NEG, jnp.float32)
        ti[...] = jnp.zeros((bc, K), jnp.int32)

    xm = x_ref[...] - bdec_ref[...]
    pre = lax.dot_general(
        xm, w_ref[...], (((1,), (1,)), ((), ())),
        preferred_element_type=jnp.float32,
    ) + benc_ref[...]

    lane = lax.broadcasted_iota(jnp.int32, (bc, BLK), 1)
    kl = lax.broadcasted_iota(jnp.int32, (bc, K), 1)

    m0 = jnp.max(pre, axis=1, keepdims=True)
    rmin0 = jnp.min(tv[...], axis=1, keepdims=True)

    def _cond(c):
        return c[2]

    def _body(c):
        work, m, _ = c
        am = jnp.min(jnp.where(work == m, lane, BLK), axis=1, keepdims=True)
        tvv = tv[...]
        rmin = jnp.min(tvv, axis=1, keepdims=True)
        rpos = jnp.min(jnp.where(tvv == rmin, kl, K), axis=1, keepdims=True)
        do = m > rmin
        sel = (kl == rpos) & do
        tv[...] = jnp.where(sel, m, tvv)
        ti[...] = jnp.where(sel, am + j * BLK, ti[...])
        work = jnp.where((lane == am) & do, _NEG, work)
        m2 = jnp.max(work, axis=1, keepdims=True)
        rmin2 = jnp.min(tv[...], axis=1, keepdims=True)
        return (work, m2, jnp.any(m2 > rmin2))

    lax.while_loop(_cond, _body, (pre, m0, jnp.any(m0 > rmin0)))

    @pl.when(j == NB - 1)
    def _fin():
        tv_out[...] = tv[...]
        ti_out[...] = ti[...]


def _encode_topk(x, W_enc, b_enc, b_dec):
    bc = x.shape[0]
    benc2 = b_enc.reshape(1, NUM_LATENTS)
    bdec2 = b_dec.reshape(1, D_IN)
    return pl.pallas_call(
        functools.partial(_encode_body, bc=bc),
        grid=(NB,),
        in_specs=[
            pl.BlockSpec((bc, D_IN), lambda j: (0, 0)),
            pl.BlockSpec((1, D_IN), lambda j: (0, 0)),
            pl.BlockSpec((BLK, D_IN), lambda j: (j, 0)),
            pl.BlockSpec((1, BLK), lambda j: (0, j)),
        ],
        out_specs=[
            pl.BlockSpec((bc, K), lambda j: (0, 0)),
            pl.BlockSpec((bc, K), lambda j: (0, 0)),
        ],
        out_shape=[
            jax.ShapeDtypeStruct((bc, K), jnp.float32),
            jax.ShapeDtypeStruct((bc, K), jnp.int32),
        ],
        scratch_shapes=[
            pltpu.VMEM((bc, K), jnp.float32),
            pltpu.VMEM((bc, K), jnp.int32),
        ],
    )(x, bdec2, W_enc, benc2)


def _make_decode(bc):
    info = plsc.get_sparse_core_info()
    NC, NS, L = info.num_cores, info.num_subcores, info.num_lanes
    NW = NC * NS
    BPW = bc // NW
    mesh = plsc.VectorSubcoreMesh(core_axis_name="c", subcore_axis_name="s")

    KH = K // 2

    @functools.partial(
        pl.kernel, mesh=mesh,
        out_type=jax.ShapeDtypeStruct((bc * D_IN,), jnp.float32),
        scratch_types=[
            pltpu.VMEM((BPW * K,), jnp.int32),
            pltpu.VMEM((BPW * K * L,), jnp.float32),
            pltpu.VMEM((KH, D_IN), jnp.float32),
            pltpu.VMEM((KH, D_IN), jnp.float32),
            pltpu.VMEM((D_IN,), jnp.float32),
            pltpu.VMEM((D_IN,), jnp.float32),
            pltpu.SemaphoreType.DMA,
            pltpu.SemaphoreType.DMA,
        ],
    )
    def _dec(acts_hbm, idx_hbm, wdec_hbm, bdec_hbm, out_hbm,
             idx_v, w_v, rows0_v, rows1_v, bdec_v, acc_v, sem0, sem1):
        wid = lax.axis_index("s") * NC + lax.axis_index("c")
        base = wid * BPW * K
        pltpu.sync_copy(bdec_hbm, bdec_v)
        pltpu.sync_copy(idx_hbm.at[pl.ds(base, BPW * K)], idx_v)
        pltpu.sync_copy(acts_hbm.at[pl.ds(base * L, BPW * K * L)], w_v)

        rows = (rows0_v, rows1_v)
        sems = (sem0, sem1)

        def _issue(s, p):
            pltpu.async_copy(
                wdec_hbm.at[idx_v.at[pl.ds(s * KH, KH)]], rows[p], sems[p])

        def _drain(p):
            pltpu.make_async_copy(
                wdec_hbm.at[pl.ds(0, KH)], rows[p], sems[p]).wait()

        _issue(0, 0)
        _issue(1, 1)

        def b_body(i, carry):
            for h in (0, 1):
                s = 2 * i + h
                _drain(h)
                ws = [w_v[pl.ds((s * KH + k) * L, L)] for k in range(KH)]

                def c_body(c, carry2, h=h, ws=ws):
                    off = c * L
                    if h == 0:
                        acc = bdec_v[pl.ds(off, L)]
                    else:
                        acc = acc_v[pl.ds(off, L)]
                    r = rows[h]
                    for k in range(KH):
                        acc = acc + ws[k] * r[k, pl.ds(off, L)]
                    acc_v[pl.ds(off, L)] = acc
                    return carry2

                lax.fori_loop(0, D_IN // L, c_body, 0)

                @pl.when(i + 1 < BPW)
                def _next(s=s, h=h):
                    _issue(s + 2, h)

            pltpu.sync_copy(
                acc_v, out_hbm.at[pl.ds((wid * BPW + i) * D_IN, D_IN)])
            return carry

        lax.fori_loop(0, BPW, b_body, 0)

    return _dec


def kernel(x, W_enc, b_enc, W_dec, b_dec):
    H = B // 2
    dec = _make_decode(H)
    parts = []
    for part in (0, 1):
        xs = lax.slice_in_dim(x, part * H, (part + 1) * H)
        top_acts, top_idx = _encode_topk(xs, W_enc, b_enc, b_dec)
        acts16 = jnp.broadcast_to(
            top_acts.reshape(H * K, 1),
            (H * K, 16)).reshape(-1).astype(jnp.float32)
        parts.append(dec(acts16, top_idx.reshape(H * K), W_dec, b_dec))
    return jnp.concatenate(parts).reshape(B, D_IN)
